# baseline (speedup 1.0000x reference)
import jax
import jax.numpy as jnp
from jax.experimental import pallas as pl
from jax.experimental.pallas import tpu as pltpu

_R = 128
_C = 4096


def _cumsum_kernel(x_ref, o_ref, carry_ref):
    r = pl.program_id(2)

    @pl.when(r == 0)
    def _():
        carry_ref[...] = jnp.zeros_like(carry_ref)

    x = x_ref[0]
    row = jax.lax.broadcasted_iota(jnp.int32, (_R, _R), 0)
    col = jax.lax.broadcasted_iota(jnp.int32, (_R, _R), 1)
    tri = (row >= col).astype(jnp.float32)
    part = jax.lax.dot(tri, x, preferred_element_type=jnp.float32)
    out = part + carry_ref[...]
    o_ref[0] = out
    carry_ref[...] = out[_R - 1:_R, :]


def kernel(inputs):
    x = inputs
    b, n, m = x.shape
    grid = (b, m // _C, n // _R)
    return pl.pallas_call(
        _cumsum_kernel,
        grid=grid,
        in_specs=[pl.BlockSpec((1, _R, _C), lambda bi, ci, ri: (bi, ri, ci))],
        out_specs=pl.BlockSpec((1, _R, _C), lambda bi, ci, ri: (bi, ri, ci)),
        out_shape=jax.ShapeDtypeStruct(x.shape, x.dtype),
        scratch_shapes=[pltpu.VMEM((1, _C), jnp.float32)],
        compiler_params=pltpu.CompilerParams(
            dimension_semantics=("parallel", "parallel", "arbitrary"),
        ),
    )(x)


# speedup vs baseline: 1.1519x; 1.1519x over previous
import jax
import jax.numpy as jnp
from jax.experimental import pallas as pl
from jax.experimental.pallas import tpu as pltpu

_R = 512
_C = 4096


def _cumsum_kernel(x_ref, o_ref, carry_ref):
    r = pl.program_id(2)

    @pl.when(r == 0)
    def _():
        carry_ref[...] = jnp.zeros_like(carry_ref)

    x = x_ref[0]
    row = jax.lax.broadcasted_iota(jnp.int32, (_R, _R), 0)
    col = jax.lax.broadcasted_iota(jnp.int32, (_R, _R), 1)
    tri = (row >= col).astype(jnp.float32)
    part = jax.lax.dot(tri, x, preferred_element_type=jnp.float32)
    out = part + carry_ref[...]
    o_ref[0] = out
    carry_ref[...] = out[_R - 1:_R, :]


def kernel(inputs):
    x = inputs
    b, n, m = x.shape
    grid = (b, m // _C, n // _R)
    return pl.pallas_call(
        _cumsum_kernel,
        grid=grid,
        in_specs=[pl.BlockSpec((1, _R, _C), lambda bi, ci, ri: (bi, ri, ci))],
        out_specs=pl.BlockSpec((1, _R, _C), lambda bi, ci, ri: (bi, ri, ci)),
        out_shape=jax.ShapeDtypeStruct(x.shape, x.dtype),
        scratch_shapes=[pltpu.VMEM((1, _C), jnp.float32)],
        compiler_params=pltpu.CompilerParams(
            dimension_semantics=("parallel", "parallel", "arbitrary"),
        ),
    )(x)
